# baseline (device time: 16025 ns/iter reference)
import jax
import jax.numpy as jnp
from jax import lax
from jax.experimental import pallas as pl
from jax.experimental.pallas import tpu as pltpu

NCHUNK = 8


def kernel(x, W, labels):
    T, D = x.shape
    V = W.shape[1]
    CV = V // NCHUNK

    def body(x_ref, w_ref, lab_ref, out_ref,
             s_acc, t_acc, send_ref, recv_ref, send_sem, recv_sem):
        i = pl.program_id(0)
        my_x = lax.axis_index("x")
        my_y = lax.axis_index("y")
        my_z = lax.axis_index("z")
        partner = (1 - my_x, my_y, my_z)
        barrier = pltpu.get_barrier_semaphore()

        @pl.when(i == 0)
        def _():
            pl.semaphore_signal(barrier, inc=1, device_id=partner,
                                device_id_type=pl.DeviceIdType.MESH)
            s_acc[:, :] = jnp.zeros((T, 1), jnp.float32)
            t_acc[:, :] = jnp.zeros((T, 1), jnp.float32)

        logits = jnp.dot(x_ref[:, :], w_ref[:, :],
                         preferred_element_type=jnp.float32)
        s_acc[:, :] += jnp.sum(jnp.exp(logits), axis=1, keepdims=True)
        col = lax.broadcasted_iota(jnp.int32, logits.shape, 1) + i * CV
        lab_local = lab_ref[:, :] - my_x * V
        t_acc[:, :] += jnp.sum(jnp.where(col == lab_local, logits, 0.0),
                               axis=1, keepdims=True)

        @pl.when(i == NCHUNK - 1)
        def _():
            s = s_acc[:, :]
            t = t_acc[:, :]
            send_ref[:, 0:1] = s
            send_ref[:, 1:2] = t
            pl.semaphore_wait(barrier, 1)
            rdma = pltpu.make_async_remote_copy(
                src_ref=send_ref, dst_ref=recv_ref,
                send_sem=send_sem, recv_sem=recv_sem,
                device_id=partner, device_id_type=pl.DeviceIdType.MESH)
            rdma.start()
            rdma.wait()
            s_p = recv_ref[:, 0:1]
            t_p = recv_ref[:, 1:2]
            out_ref[:, :] = jnp.log(s + s_p) - (t + t_p)

    out = pl.pallas_call(
        body,
        grid=(NCHUNK,),
        out_shape=jax.ShapeDtypeStruct((T, 1), jnp.float32),
        in_specs=[
            pl.BlockSpec((T, D), lambda i: (0, 0), memory_space=pltpu.VMEM),
            pl.BlockSpec((D, CV), lambda i: (0, i), memory_space=pltpu.VMEM),
            pl.BlockSpec((T, 1), lambda i: (0, 0), memory_space=pltpu.VMEM),
        ],
        out_specs=pl.BlockSpec((T, 1), lambda i: (0, 0),
                               memory_space=pltpu.VMEM),
        scratch_shapes=[
            pltpu.VMEM((T, 1), jnp.float32),
            pltpu.VMEM((T, 1), jnp.float32),
            pltpu.VMEM((T, 2), jnp.float32),
            pltpu.VMEM((T, 2), jnp.float32),
            pltpu.SemaphoreType.DMA,
            pltpu.SemaphoreType.DMA,
        ],
        compiler_params=pltpu.CompilerParams(collective_id=0),
    )(x, W, labels.reshape(T, 1))
    return out.reshape(T)
